# baseline (speedup 1.0000x reference)
import functools
import math

import jax
import jax.numpy as jnp
from jax import lax
from jax.experimental import pallas as pl
from jax.experimental.pallas import tpu as pltpu
from jax.experimental.pallas import tpu_sc as plsc

N = 10000
E = 320000
D = 128
G = 128
INV_BN = 1.0 / math.sqrt(1.0 + 1e-5)

NC, NS = 2, 16
NW = NC * NS
E_PER_TILE = E // NW
CH = 104
N_CHUNKS = 96
TAIL = E_PER_TILE - N_CHUNKS * CH
N_PAD = 10240
ROWS_PER_TILE = N_PAD // NS


def _sc_scatter(h, src1, dst1):
    mesh = plsc.VectorSubcoreMesh(core_axis_name="c", subcore_axis_name="s")

    @functools.partial(
        pl.kernel,
        out_type=jax.ShapeDtypeStruct((NC, N_PAD, D), jnp.float32),
        mesh=mesh,
        scratch_types=[
            pltpu.VMEM((CH,), jnp.int32),
            pltpu.VMEM((CH,), jnp.int32),
            pltpu.VMEM((CH,), jnp.int32),
            pltpu.VMEM((CH,), jnp.int32),
            pltpu.VMEM((CH,), jnp.int32),
            pltpu.VMEM((CH,), jnp.int32),
            pltpu.VMEM((CH, D), jnp.float32),
            pltpu.VMEM((CH, D), jnp.float32),
            pltpu.VMEM((CH, D), jnp.float32),
            pltpu.VMEM((TAIL,), jnp.int32),
            pltpu.VMEM((TAIL,), jnp.int32),
            pltpu.VMEM((TAIL, D), jnp.float32),
            pltpu.VMEM_SHARED((N_PAD, D), jnp.float32),
            pltpu.SemaphoreType.DMA,
            pltpu.SemaphoreType.DMA,
            pltpu.SemaphoreType.DMA,
            pltpu.SemaphoreType.DMA,
            pltpu.SemaphoreType.DMA,
            pltpu.SemaphoreType.DMA,
        ],
    )
    def k(h_hbm, src_hbm, dst_hbm, out_hbm, src_a, src_b, src_c,
          dst_a, dst_b, dst_c, rows_a, rows_b, rows_c, src_t, dst_t, rows_t,
          acc_sh, sem_a, sem_b, sem_c, isem_a, isem_b, isem_c):
        c = lax.axis_index("c")
        s = lax.axis_index("s")
        wid = s * NC + c
        ebase = wid * E_PER_TILE
        bufs = ((src_a, dst_a, rows_a, sem_a, isem_a),
                (src_b, dst_b, rows_b, sem_b, isem_b),
                (src_c, dst_c, rows_c, sem_c, isem_c))

        def fire_idx(chunk, b):
            srcb, dstb, _, _, isem = b
            off = pl.multiple_of(ebase + chunk * CH, 8)
            pltpu.async_copy(src_hbm.at[pl.ds(off, CH)], srcb, isem)
            pltpu.async_copy(dst_hbm.at[pl.ds(off, CH)], dstb, isem)

        def fire_gather(b):
            srcb, _, rows, sem, isem = b
            pltpu.make_async_copy(src_hbm.at[pl.ds(0, CH)], srcb, isem).wait()
            pltpu.make_async_copy(dst_hbm.at[pl.ds(0, CH)],
                                  b[1], isem).wait()
            pltpu.async_copy(h_hbm.at[srcb], rows, sem)

        def drain_scat(b):
            srcb, dstb, rows, sem, _ = b
            pltpu.make_async_copy(h_hbm.at[srcb], rows, sem).wait()
            pltpu.sync_copy(rows, acc_sh.at[dstb], add=True)

        for j in range(3):
            fire_idx(j, bufs[j])

        zvec = jnp.zeros((16,), jnp.float32)

        def zrow(i, carry):
            for j in range(D // 16):
                rows_a[i, pl.ds(j * 16, 16)] = zvec
            return carry

        lax.fori_loop(0, CH, zrow, 0)
        row0 = s * ROWS_PER_TILE
        nz = ROWS_PER_TILE // CH
        rz = ROWS_PER_TILE - nz * CH
        for r in range(nz):
            pltpu.sync_copy(rows_a, acc_sh.at[pl.ds(row0 + r * CH, CH)])
        pltpu.sync_copy(rows_a.at[pl.ds(0, rz)],
                        acc_sh.at[pl.ds(row0 + nz * CH, rz)])
        for j in range(3):
            fire_gather(bufs[j])
        plsc.subcore_barrier()

        def body(kk, carry):
            base = 3 * kk
            for j in range(3):
                drain_scat(bufs[j])
                fire_idx(base + 3 + j, bufs[j])
            for j in range(3):
                fire_gather(bufs[j])
            return carry

        lax.fori_loop(0, (N_CHUNKS - 6) // 3, body, 0)
        for j in range(3):
            drain_scat(bufs[j])
            fire_idx(N_CHUNKS - 3 + j, bufs[j])
        for j in range(3):
            fire_gather(bufs[j])
        for j in range(3):
            drain_scat(bufs[j])
        toff = pl.multiple_of(ebase + N_CHUNKS * CH, 8)
        pltpu.async_copy(src_hbm.at[pl.ds(toff, TAIL)], src_t, isem_a)
        pltpu.async_copy(dst_hbm.at[pl.ds(toff, TAIL)], dst_t, isem_a)
        pltpu.make_async_copy(src_hbm.at[pl.ds(0, TAIL)], src_t, isem_a).wait()
        pltpu.make_async_copy(dst_hbm.at[pl.ds(0, TAIL)], dst_t, isem_a).wait()
        pltpu.async_copy(h_hbm.at[src_t], rows_t, sem_a)
        pltpu.make_async_copy(h_hbm.at[src_t], rows_t, sem_a).wait()
        pltpu.sync_copy(rows_t, acc_sh.at[dst_t], add=True)
        plsc.subcore_barrier()

        pltpu.sync_copy(acc_sh.at[pl.ds(row0, ROWS_PER_TILE)],
                        out_hbm.at[c, pl.ds(row0, ROWS_PER_TILE)])

    return k(h, src1, dst1)


RB = 2000
N_RB = N // RB


def _tc_layer(h, agg, batch3d, W1, b1, W2, b2, prev_pooled=None):
    final = prev_pooled is not None

    def body(h_ref, a_ref, b_ref, W1_ref, b1_ref, W2_ref, b2_ref,
             *rest):
        if final:
            p1_ref, p2_ref, o_ref, p_ref = rest
        else:
            o_ref, p_ref = rest
        i = pl.program_id(0)

        @pl.when(i == 0)
        def _():
            if final:
                p_ref[:, 0:D] = p1_ref[...]
                p_ref[:, D:2 * D] = p2_ref[...]
                p_ref[:, 2 * D:3 * D] = jnp.zeros((G, D), jnp.float32)
            else:
                p_ref[...] = jnp.zeros_like(p_ref)

        m = a_ref[0] + a_ref[1] + h_ref[...]
        z = jnp.maximum(
            jnp.dot(m, W1_ref[...], preferred_element_type=jnp.float32)
            + b1_ref[...], 0.0)
        o = jnp.maximum(
            jnp.dot(z, W2_ref[...], preferred_element_type=jnp.float32)
            + b2_ref[...], 0.0) * INV_BN
        o_ref[...] = o
        sel = (lax.broadcasted_iota(jnp.int32, (G, RB), 0)
               == b_ref[...].reshape(1, RB)).astype(jnp.float32)
        pool = jnp.dot(sel, o, preferred_element_type=jnp.float32)
        if final:
            p_ref[:, 2 * D:3 * D] += pool
        else:
            p_ref[...] += pool

    gd = pl.BlockSpec((G, D), lambda i: (0, 0))
    in_specs = [
        pl.BlockSpec((RB, D), lambda i: (i, 0)),
        pl.BlockSpec((2, RB, D), lambda i: (0, i, 0)),
        pl.BlockSpec((1, 1, RB), lambda i: (i, 0, 0)),
        pl.BlockSpec((D, D), lambda i: (0, 0)),
        pl.BlockSpec((1, D), lambda i: (0, 0)),
        pl.BlockSpec((D, D), lambda i: (0, 0)),
        pl.BlockSpec((1, D), lambda i: (0, 0)),
    ]
    args = [h, agg, batch3d, W1, b1, W2, b2]
    pw = 3 * D if final else D
    if final:
        in_specs += [gd, gd]
        args += list(prev_pooled)
    return pl.pallas_call(
        body,
        grid=(N_RB,),
        in_specs=in_specs,
        out_specs=[
            pl.BlockSpec((RB, D), lambda i: (i, 0)),
            pl.BlockSpec((G, pw), lambda i: (0, 0)),
        ],
        out_shape=[
            jax.ShapeDtypeStruct((N, D), jnp.float32),
            jax.ShapeDtypeStruct((G, pw), jnp.float32),
        ],
    )(*args)


def kernel(x, edge_index, batch, num_graphs, W1_0, b1_0, W2_0, b2_0,
           W1_1, b1_1, W2_1, b2_1, W1_2, b1_2, W2_2, b2_2):
    src1 = edge_index[0]
    dst1 = edge_index[1]
    batch3d = batch.reshape(N_RB, 1, RB)
    params = [(W1_0, b1_0, W2_0, b2_0), (W1_1, b1_1, W2_1, b2_1),
              (W1_2, b1_2, W2_2, b2_2)]
    h = x
    pooled = []
    for li, (W1, b1, W2, b2) in enumerate(params):
        agg = _sc_scatter(h, src1, dst1)
        prev = tuple(pooled) if li == 2 else None
        h, p = _tc_layer(h, agg, batch3d,
                         W1, b1.reshape(1, D), W2, b2.reshape(1, D),
                         prev_pooled=prev)
        pooled.append(p)
    return pooled[2]


# speedup vs baseline: 1.2833x; 1.2833x over previous
import functools
import math

import jax
import jax.numpy as jnp
from jax import lax
from jax.experimental import pallas as pl
from jax.experimental.pallas import tpu as pltpu
from jax.experimental.pallas import tpu_sc as plsc

N = 10000
E = 320000
D = 128
G = 128
INV_BN = 1.0 / math.sqrt(1.0 + 1e-5)

NC, NS = 2, 16
NW = NC * NS
E_PER_TILE = E // NW
CH = 128
N_CHUNKS = E_PER_TILE // CH
TAIL = E_PER_TILE - N_CHUNKS * CH
N_PAD = 10240
ROWS_PER_TILE = N_PAD // NS


def _sc_scatter(h, src2, dst1):
    mesh = plsc.VectorSubcoreMesh(core_axis_name="c", subcore_axis_name="s")

    @functools.partial(
        pl.kernel,
        out_type=jax.ShapeDtypeStruct((NC, N_PAD, D), jnp.float32),
        mesh=mesh,
        scratch_types=[
            pltpu.VMEM((E_PER_TILE,), jnp.int32),
            pltpu.VMEM((CH,), jnp.int32),
            pltpu.VMEM((CH,), jnp.int32),
            pltpu.VMEM((TAIL,), jnp.int32),
            pltpu.VMEM((CH, D), jnp.float32),
            pltpu.VMEM((CH, D), jnp.float32),
            pltpu.VMEM((TAIL, D), jnp.float32),
            pltpu.VMEM_SHARED((N_PAD, D), jnp.float32),
            pltpu.SemaphoreType.DMA,
            pltpu.SemaphoreType.DMA,
            pltpu.SemaphoreType.DMA,
            pltpu.SemaphoreType.DMA,
        ],
    )
    def k(h_hbm, src_hbm, dst_hbm, out_hbm, src_v, dst_a, dst_b, dst_t,
          rows_a, rows_b, rows_t, acc_sh, sem_a, sem_b, sem_da, sem_db):
        c = lax.axis_index("c")
        s = lax.axis_index("s")
        wid = s * NC + c
        ebase = wid * E_PER_TILE

        stage = pltpu.make_async_copy(src_hbm.at[wid], src_v, sem_da)
        stage.start()

        def fire(chunk, rows, sem, dstb, dsem):
            off = pl.multiple_of(chunk * CH, 8)
            pltpu.async_copy(dst_hbm.at[pl.ds(ebase + off, CH)], dstb, dsem)
            pltpu.async_copy(h_hbm.at[src_v.at[pl.ds(off, CH)]], rows, sem)

        def drain(rows, sem, dstb, dsem):
            pltpu.make_async_copy(dst_hbm.at[pl.ds(0, CH)], dstb, dsem).wait()
            pltpu.make_async_copy(h_hbm.at[src_v.at[pl.ds(0, CH)]],
                                  rows, sem).wait()

        def scat(rows, dstb):
            pltpu.sync_copy(rows, acc_sh.at[dstb], add=True)

        zvec = jnp.zeros((16,), jnp.float32)

        def zrow(i, carry):
            for j in range(D // 16):
                rows_b[i, pl.ds(j * 16, 16)] = zvec
            return carry

        lax.fori_loop(0, CH, zrow, 0)
        stage.wait()
        fire(0, rows_a, sem_a, dst_a, sem_da)
        row0 = s * ROWS_PER_TILE
        for r in range(ROWS_PER_TILE // CH):
            pltpu.sync_copy(rows_b, acc_sh.at[pl.ds(row0 + r * CH, CH)])
        fire(1, rows_b, sem_b, dst_b, sem_db)
        plsc.subcore_barrier()

        def body(kk, carry):
            drain(rows_a, sem_a, dst_a, sem_da)
            scat(rows_a, dst_a)
            fire(2 * kk + 2, rows_a, sem_a, dst_a, sem_da)
            drain(rows_b, sem_b, dst_b, sem_db)
            scat(rows_b, dst_b)
            fire(2 * kk + 3, rows_b, sem_b, dst_b, sem_db)
            return carry

        lax.fori_loop(0, (N_CHUNKS - 4) // 2, body, 0)
        drain(rows_a, sem_a, dst_a, sem_da)
        scat(rows_a, dst_a)
        fire(N_CHUNKS - 2, rows_a, sem_a, dst_a, sem_da)
        drain(rows_b, sem_b, dst_b, sem_db)
        scat(rows_b, dst_b)
        fire(N_CHUNKS - 1, rows_b, sem_b, dst_b, sem_db)
        drain(rows_a, sem_a, dst_a, sem_da)
        scat(rows_a, dst_a)
        drain(rows_b, sem_b, dst_b, sem_db)
        scat(rows_b, dst_b)
        toff = N_CHUNKS * CH
        pltpu.async_copy(dst_hbm.at[pl.ds(ebase + toff, TAIL)], dst_t, sem_da)
        pltpu.async_copy(h_hbm.at[src_v.at[pl.ds(toff, TAIL)]], rows_t, sem_a)
        pltpu.make_async_copy(dst_hbm.at[pl.ds(0, TAIL)], dst_t, sem_da).wait()
        pltpu.make_async_copy(h_hbm.at[src_v.at[pl.ds(0, TAIL)]],
                              rows_t, sem_a).wait()
        pltpu.sync_copy(rows_t, acc_sh.at[dst_t], add=True)
        plsc.subcore_barrier()

        pltpu.sync_copy(acc_sh.at[pl.ds(row0, ROWS_PER_TILE)],
                        out_hbm.at[c, pl.ds(row0, ROWS_PER_TILE)])

    return k(h, src2, dst1)


RB = 2000
N_RB = N // RB


def _tc_layer(h, agg, batch3d, W1, b1, W2, b2, prev_pooled=None):
    final = prev_pooled is not None

    def body(h_ref, a_ref, b_ref, W1_ref, b1_ref, W2_ref, b2_ref,
             *rest):
        if final:
            p1_ref, p2_ref, o_ref, p_ref = rest
        else:
            o_ref, p_ref = rest
        i = pl.program_id(0)

        @pl.when(i == 0)
        def _():
            if final:
                p_ref[:, 0:D] = p1_ref[...]
                p_ref[:, D:2 * D] = p2_ref[...]
                p_ref[:, 2 * D:3 * D] = jnp.zeros((G, D), jnp.float32)
            else:
                p_ref[...] = jnp.zeros_like(p_ref)

        m = a_ref[0] + a_ref[1] + h_ref[...]
        z = jnp.maximum(
            jnp.dot(m, W1_ref[...], preferred_element_type=jnp.float32)
            + b1_ref[...], 0.0)
        o = jnp.maximum(
            jnp.dot(z, W2_ref[...], preferred_element_type=jnp.float32)
            + b2_ref[...], 0.0) * INV_BN
        o_ref[...] = o
        sel = (lax.broadcasted_iota(jnp.int32, (G, RB), 0)
               == b_ref[...].reshape(1, RB)).astype(jnp.float32)
        pool = jnp.dot(sel, o, preferred_element_type=jnp.float32)
        if final:
            p_ref[:, 2 * D:3 * D] += pool
        else:
            p_ref[...] += pool

    gd = pl.BlockSpec((G, D), lambda i: (0, 0))
    in_specs = [
        pl.BlockSpec((RB, D), lambda i: (i, 0)),
        pl.BlockSpec((2, RB, D), lambda i: (0, i, 0)),
        pl.BlockSpec((1, 1, RB), lambda i: (i, 0, 0)),
        pl.BlockSpec((D, D), lambda i: (0, 0)),
        pl.BlockSpec((1, D), lambda i: (0, 0)),
        pl.BlockSpec((D, D), lambda i: (0, 0)),
        pl.BlockSpec((1, D), lambda i: (0, 0)),
    ]
    args = [h, agg, batch3d, W1, b1, W2, b2]
    pw = 3 * D if final else D
    if final:
        in_specs += [gd, gd]
        args += list(prev_pooled)
    return pl.pallas_call(
        body,
        grid=(N_RB,),
        in_specs=in_specs,
        out_specs=[
            pl.BlockSpec((RB, D), lambda i: (i, 0)),
            pl.BlockSpec((G, pw), lambda i: (0, 0)),
        ],
        out_shape=[
            jax.ShapeDtypeStruct((N, D), jnp.float32),
            jax.ShapeDtypeStruct((G, pw), jnp.float32),
        ],
    )(*args)


def kernel(x, edge_index, batch, num_graphs, W1_0, b1_0, W2_0, b2_0,
           W1_1, b1_1, W2_1, b2_1, W1_2, b1_2, W2_2, b2_2):
    src2 = edge_index[0].reshape(NW, E_PER_TILE)
    dst1 = edge_index[1]
    batch3d = batch.reshape(N_RB, 1, RB)
    params = [(W1_0, b1_0, W2_0, b2_0), (W1_1, b1_1, W2_1, b2_1),
              (W1_2, b1_2, W2_2, b2_2)]
    h = x
    pooled = []
    for li, (W1, b1, W2, b2) in enumerate(params):
        agg = _sc_scatter(h, src2, dst1)
        prev = tuple(pooled) if li == 2 else None
        h, p = _tc_layer(h, agg, batch3d,
                         W1, b1.reshape(1, D), W2, b2.reshape(1, D),
                         prev_pooled=prev)
        pooled.append(p)
    return pooled[2]


# speedup vs baseline: 1.2877x; 1.0034x over previous
import functools
import math

import jax
import jax.numpy as jnp
from jax import lax
from jax.experimental import pallas as pl
from jax.experimental.pallas import tpu as pltpu
from jax.experimental.pallas import tpu_sc as plsc

N = 10000
E = 320000
D = 128
G = 128
INV_BN = 1.0 / math.sqrt(1.0 + 1e-5)

NC, NS = 2, 16
NW = NC * NS
E_PER_TILE = E // NW
CH = 128
N_CHUNKS = E_PER_TILE // CH
TAIL = E_PER_TILE - N_CHUNKS * CH
N_PAD = 10240
ROWS_PER_TILE = N_PAD // NS


def _sc_scatter(h, src2, dst1):
    mesh = plsc.VectorSubcoreMesh(core_axis_name="c", subcore_axis_name="s")

    @functools.partial(
        pl.kernel,
        out_type=jax.ShapeDtypeStruct((NC, N_PAD, D), jnp.float32),
        mesh=mesh,
        scratch_types=[
            pltpu.VMEM((E_PER_TILE,), jnp.int32),
            pltpu.VMEM((CH,), jnp.int32),
            pltpu.VMEM((CH,), jnp.int32),
            pltpu.VMEM((TAIL,), jnp.int32),
            pltpu.VMEM((CH, D), jnp.float32),
            pltpu.VMEM((CH, D), jnp.float32),
            pltpu.VMEM((TAIL, D), jnp.float32),
            pltpu.VMEM_SHARED((N_PAD, D), jnp.float32),
            pltpu.SemaphoreType.DMA,
            pltpu.SemaphoreType.DMA,
            pltpu.SemaphoreType.DMA,
            pltpu.SemaphoreType.DMA,
        ],
    )
    def k(h_hbm, src_hbm, dst_hbm, out_hbm, src_v, dst_a, dst_b, dst_t,
          rows_a, rows_b, rows_t, acc_sh, sem_a, sem_b, sem_da, sem_db):
        c = lax.axis_index("c")
        s = lax.axis_index("s")
        wid = s * NC + c
        ebase = wid * E_PER_TILE

        stage = pltpu.make_async_copy(src_hbm.at[wid], src_v, sem_da)
        stage.start()

        def fire(chunk, rows, sem, dstb, dsem):
            off = pl.multiple_of(chunk * CH, 8)
            pltpu.async_copy(dst_hbm.at[pl.ds(ebase + off, CH)], dstb, dsem)
            pltpu.async_copy(h_hbm.at[src_v.at[pl.ds(off, CH)]], rows, sem)

        def drain(rows, sem, dstb, dsem):
            pltpu.make_async_copy(dst_hbm.at[pl.ds(0, CH)], dstb, dsem).wait()
            pltpu.make_async_copy(h_hbm.at[src_v.at[pl.ds(0, CH)]],
                                  rows, sem).wait()

        def scat(rows, dstb):
            pltpu.sync_copy(rows, acc_sh.at[dstb], add=True)

        zvec = jnp.zeros((16,), jnp.float32)

        def zrow(i, carry):
            for j in range(D // 16):
                rows_b[i, pl.ds(j * 16, 16)] = zvec
            return carry

        lax.fori_loop(0, CH, zrow, 0)
        stage.wait()
        fire(0, rows_a, sem_a, dst_a, sem_da)
        row0 = s * ROWS_PER_TILE
        zcops = [pltpu.make_async_copy(
            rows_b, acc_sh.at[pl.ds(row0 + r * CH, CH)], sem_db)
            for r in range(ROWS_PER_TILE // CH)]
        for z in zcops:
            z.start()
        for z in zcops:
            z.wait()
        fire(1, rows_b, sem_b, dst_b, sem_db)
        plsc.subcore_barrier()

        def body(kk, carry):
            drain(rows_a, sem_a, dst_a, sem_da)
            scat(rows_a, dst_a)
            fire(2 * kk + 2, rows_a, sem_a, dst_a, sem_da)
            drain(rows_b, sem_b, dst_b, sem_db)
            scat(rows_b, dst_b)
            fire(2 * kk + 3, rows_b, sem_b, dst_b, sem_db)
            return carry

        lax.fori_loop(0, (N_CHUNKS - 4) // 2, body, 0)
        drain(rows_a, sem_a, dst_a, sem_da)
        scat(rows_a, dst_a)
        fire(N_CHUNKS - 2, rows_a, sem_a, dst_a, sem_da)
        drain(rows_b, sem_b, dst_b, sem_db)
        scat(rows_b, dst_b)
        fire(N_CHUNKS - 1, rows_b, sem_b, dst_b, sem_db)
        drain(rows_a, sem_a, dst_a, sem_da)
        scat(rows_a, dst_a)
        drain(rows_b, sem_b, dst_b, sem_db)
        scat(rows_b, dst_b)
        toff = N_CHUNKS * CH
        pltpu.async_copy(dst_hbm.at[pl.ds(ebase + toff, TAIL)], dst_t, sem_da)
        pltpu.async_copy(h_hbm.at[src_v.at[pl.ds(toff, TAIL)]], rows_t, sem_a)
        pltpu.make_async_copy(dst_hbm.at[pl.ds(0, TAIL)], dst_t, sem_da).wait()
        pltpu.make_async_copy(h_hbm.at[src_v.at[pl.ds(0, TAIL)]],
                              rows_t, sem_a).wait()
        pltpu.sync_copy(rows_t, acc_sh.at[dst_t], add=True)
        plsc.subcore_barrier()

        pltpu.sync_copy(acc_sh.at[pl.ds(row0, ROWS_PER_TILE)],
                        out_hbm.at[c, pl.ds(row0, ROWS_PER_TILE)])

    return k(h, src2, dst1)


RB = 2000
N_RB = N // RB


def _tc_layer(h, agg, batch3d, W1, b1, W2, b2, prev_pooled=None):
    final = prev_pooled is not None

    def body(h_ref, a_ref, b_ref, W1_ref, b1_ref, W2_ref, b2_ref,
             *rest):
        if final:
            p1_ref, p2_ref, o_ref, p_ref = rest
        else:
            o_ref, p_ref = rest
        i = pl.program_id(0)

        @pl.when(i == 0)
        def _():
            if final:
                p_ref[:, 0:D] = p1_ref[...]
                p_ref[:, D:2 * D] = p2_ref[...]
                p_ref[:, 2 * D:3 * D] = jnp.zeros((G, D), jnp.float32)
            else:
                p_ref[...] = jnp.zeros_like(p_ref)

        m = a_ref[0] + a_ref[1] + h_ref[...]
        z = jnp.maximum(
            jnp.dot(m, W1_ref[...], preferred_element_type=jnp.float32)
            + b1_ref[...], 0.0)
        o = jnp.maximum(
            jnp.dot(z, W2_ref[...], preferred_element_type=jnp.float32)
            + b2_ref[...], 0.0) * INV_BN
        o_ref[...] = o
        sel = (lax.broadcasted_iota(jnp.int32, (G, RB), 0)
               == b_ref[...].reshape(1, RB)).astype(jnp.float32)
        pool = jnp.dot(sel, o, preferred_element_type=jnp.float32)
        if final:
            p_ref[:, 2 * D:3 * D] += pool
        else:
            p_ref[...] += pool

    gd = pl.BlockSpec((G, D), lambda i: (0, 0))
    in_specs = [
        pl.BlockSpec((RB, D), lambda i: (i, 0)),
        pl.BlockSpec((2, RB, D), lambda i: (0, i, 0)),
        pl.BlockSpec((1, 1, RB), lambda i: (i, 0, 0)),
        pl.BlockSpec((D, D), lambda i: (0, 0)),
        pl.BlockSpec((1, D), lambda i: (0, 0)),
        pl.BlockSpec((D, D), lambda i: (0, 0)),
        pl.BlockSpec((1, D), lambda i: (0, 0)),
    ]
    args = [h, agg, batch3d, W1, b1, W2, b2]
    pw = 3 * D if final else D
    if final:
        in_specs += [gd, gd]
        args += list(prev_pooled)
    return pl.pallas_call(
        body,
        grid=(N_RB,),
        in_specs=in_specs,
        out_specs=[
            pl.BlockSpec((RB, D), lambda i: (i, 0)),
            pl.BlockSpec((G, pw), lambda i: (0, 0)),
        ],
        out_shape=[
            jax.ShapeDtypeStruct((N, D), jnp.float32),
            jax.ShapeDtypeStruct((G, pw), jnp.float32),
        ],
    )(*args)


def kernel(x, edge_index, batch, num_graphs, W1_0, b1_0, W2_0, b2_0,
           W1_1, b1_1, W2_1, b2_1, W1_2, b1_2, W2_2, b2_2):
    src2 = edge_index[0].reshape(NW, E_PER_TILE)
    dst1 = edge_index[1]
    batch3d = batch.reshape(N_RB, 1, RB)
    params = [(W1_0, b1_0, W2_0, b2_0), (W1_1, b1_1, W2_1, b2_1),
              (W1_2, b1_2, W2_2, b2_2)]
    h = x
    pooled = []
    for li, (W1, b1, W2, b2) in enumerate(params):
        agg = _sc_scatter(h, src2, dst1)
        prev = tuple(pooled) if li == 2 else None
        h, p = _tc_layer(h, agg, batch3d,
                         W1, b1.reshape(1, D), W2, b2.reshape(1, D),
                         prev_pooled=prev)
        pooled.append(p)
    return pooled[2]


# speedup vs baseline: 1.2910x; 1.0026x over previous
import functools
import math

import jax
import jax.numpy as jnp
from jax import lax
from jax.experimental import pallas as pl
from jax.experimental.pallas import tpu as pltpu
from jax.experimental.pallas import tpu_sc as plsc

N = 10000
E = 320000
D = 128
G = 128
INV_BN = 1.0 / math.sqrt(1.0 + 1e-5)

NC, NS = 2, 16
NW = NC * NS
E_PER_TILE = E // NW
CH = 128
N_CHUNKS = E_PER_TILE // CH
TAIL = E_PER_TILE - N_CHUNKS * CH
N_PAD = 10240
ROWS_PER_TILE = N_PAD // NS


def _sc_scatter(h, src2, dst1):
    mesh = plsc.VectorSubcoreMesh(core_axis_name="c", subcore_axis_name="s")

    @functools.partial(
        pl.kernel,
        out_type=jax.ShapeDtypeStruct((NC, N_PAD, D), jnp.float32),
        mesh=mesh,
        scratch_types=[
            pltpu.VMEM((E_PER_TILE,), jnp.int32),
            pltpu.VMEM((CH,), jnp.int32),
            pltpu.VMEM((CH,), jnp.int32),
            pltpu.VMEM((TAIL,), jnp.int32),
            pltpu.VMEM((CH, D), jnp.float32),
            pltpu.VMEM((CH, D), jnp.float32),
            pltpu.VMEM((TAIL, D), jnp.float32),
            pltpu.VMEM_SHARED((N_PAD, D), jnp.float32),
            pltpu.SemaphoreType.DMA,
            pltpu.SemaphoreType.DMA,
            pltpu.SemaphoreType.DMA,
            pltpu.SemaphoreType.DMA,
            pltpu.SemaphoreType.DMA,
        ],
    )
    def k(h_hbm, src_hbm, dst_hbm, out_hbm, src_v, dst_a, dst_b, dst_t,
          rows_a, rows_b, rows_t, acc_sh, sem_a, sem_b, sem_da, sem_db,
          sem_t):
        c = lax.axis_index("c")
        s = lax.axis_index("s")
        wid = s * NC + c
        ebase = wid * E_PER_TILE

        stage = pltpu.make_async_copy(src_hbm.at[wid], src_v, sem_da)
        stage.start()

        def fire(chunk, rows, sem, dstb, dsem):
            off = pl.multiple_of(chunk * CH, 8)
            pltpu.async_copy(dst_hbm.at[pl.ds(ebase + off, CH)], dstb, dsem)
            pltpu.async_copy(h_hbm.at[src_v.at[pl.ds(off, CH)]], rows, sem)

        def drain(rows, sem, dstb, dsem):
            pltpu.make_async_copy(dst_hbm.at[pl.ds(0, CH)], dstb, dsem).wait()
            pltpu.make_async_copy(h_hbm.at[src_v.at[pl.ds(0, CH)]],
                                  rows, sem).wait()

        def scat(rows, dstb):
            pltpu.sync_copy(rows, acc_sh.at[dstb], add=True)

        zvec = jnp.zeros((16,), jnp.float32)

        def zrow(i, carry):
            for j in range(D // 16):
                rows_b[i, pl.ds(j * 16, 16)] = zvec
            return carry

        lax.fori_loop(0, CH, zrow, 0)
        stage.wait()
        fire(0, rows_a, sem_a, dst_a, sem_da)
        row0 = s * ROWS_PER_TILE
        zcops = [pltpu.make_async_copy(
            rows_b, acc_sh.at[pl.ds(row0 + r * CH, CH)], sem_db)
            for r in range(ROWS_PER_TILE // CH)]
        for z in zcops:
            z.start()
        for z in zcops:
            z.wait()
        fire(1, rows_b, sem_b, dst_b, sem_db)
        toff = N_CHUNKS * CH
        pltpu.async_copy(dst_hbm.at[pl.ds(ebase + toff, TAIL)], dst_t, sem_t)
        pltpu.async_copy(h_hbm.at[src_v.at[pl.ds(toff, TAIL)]], rows_t, sem_t)
        plsc.subcore_barrier()

        def body(kk, carry):
            drain(rows_a, sem_a, dst_a, sem_da)
            scat(rows_a, dst_a)
            fire(2 * kk + 2, rows_a, sem_a, dst_a, sem_da)
            drain(rows_b, sem_b, dst_b, sem_db)
            scat(rows_b, dst_b)
            fire(2 * kk + 3, rows_b, sem_b, dst_b, sem_db)
            return carry

        lax.fori_loop(0, (N_CHUNKS - 4) // 2, body, 0)
        drain(rows_a, sem_a, dst_a, sem_da)
        scat(rows_a, dst_a)
        fire(N_CHUNKS - 2, rows_a, sem_a, dst_a, sem_da)
        drain(rows_b, sem_b, dst_b, sem_db)
        scat(rows_b, dst_b)
        fire(N_CHUNKS - 1, rows_b, sem_b, dst_b, sem_db)
        drain(rows_a, sem_a, dst_a, sem_da)
        scat(rows_a, dst_a)
        drain(rows_b, sem_b, dst_b, sem_db)
        scat(rows_b, dst_b)
        pltpu.make_async_copy(dst_hbm.at[pl.ds(0, TAIL)], dst_t, sem_t).wait()
        pltpu.make_async_copy(h_hbm.at[src_v.at[pl.ds(0, TAIL)]],
                              rows_t, sem_t).wait()
        pltpu.sync_copy(rows_t, acc_sh.at[dst_t], add=True)
        plsc.subcore_barrier()

        pltpu.sync_copy(acc_sh.at[pl.ds(row0, ROWS_PER_TILE)],
                        out_hbm.at[c, pl.ds(row0, ROWS_PER_TILE)])

    return k(h, src2, dst1)


RB = 2000
N_RB = N // RB


def _tc_layer(h, agg, batch3d, W1, b1, W2, b2, prev_pooled=None):
    final = prev_pooled is not None

    def body(h_ref, a_ref, b_ref, W1_ref, b1_ref, W2_ref, b2_ref,
             *rest):
        if final:
            p1_ref, p2_ref, o_ref, p_ref = rest
        else:
            o_ref, p_ref = rest
        i = pl.program_id(0)

        @pl.when(i == 0)
        def _():
            if final:
                p_ref[:, 0:D] = p1_ref[...]
                p_ref[:, D:2 * D] = p2_ref[...]
                p_ref[:, 2 * D:3 * D] = jnp.zeros((G, D), jnp.float32)
            else:
                p_ref[...] = jnp.zeros_like(p_ref)

        m = a_ref[0] + a_ref[1] + h_ref[...]
        z = jnp.maximum(
            jnp.dot(m, W1_ref[...], preferred_element_type=jnp.float32)
            + b1_ref[...], 0.0)
        o = jnp.maximum(
            jnp.dot(z, W2_ref[...], preferred_element_type=jnp.float32)
            + b2_ref[...], 0.0) * INV_BN
        o_ref[...] = o
        sel = (lax.broadcasted_iota(jnp.int32, (G, RB), 0)
               == b_ref[...].reshape(1, RB)).astype(jnp.float32)
        pool = jnp.dot(sel, o, preferred_element_type=jnp.float32)
        if final:
            p_ref[:, 2 * D:3 * D] += pool
        else:
            p_ref[...] += pool

    gd = pl.BlockSpec((G, D), lambda i: (0, 0))
    in_specs = [
        pl.BlockSpec((RB, D), lambda i: (i, 0)),
        pl.BlockSpec((2, RB, D), lambda i: (0, i, 0)),
        pl.BlockSpec((1, 1, RB), lambda i: (i, 0, 0)),
        pl.BlockSpec((D, D), lambda i: (0, 0)),
        pl.BlockSpec((1, D), lambda i: (0, 0)),
        pl.BlockSpec((D, D), lambda i: (0, 0)),
        pl.BlockSpec((1, D), lambda i: (0, 0)),
    ]
    args = [h, agg, batch3d, W1, b1, W2, b2]
    pw = 3 * D if final else D
    if final:
        in_specs += [gd, gd]
        args += list(prev_pooled)
    return pl.pallas_call(
        body,
        grid=(N_RB,),
        in_specs=in_specs,
        out_specs=[
            pl.BlockSpec((RB, D), lambda i: (i, 0)),
            pl.BlockSpec((G, pw), lambda i: (0, 0)),
        ],
        out_shape=[
            jax.ShapeDtypeStruct((N, D), jnp.float32),
            jax.ShapeDtypeStruct((G, pw), jnp.float32),
        ],
    )(*args)


def kernel(x, edge_index, batch, num_graphs, W1_0, b1_0, W2_0, b2_0,
           W1_1, b1_1, W2_1, b2_1, W1_2, b1_2, W2_2, b2_2):
    src2 = edge_index[0].reshape(NW, E_PER_TILE)
    dst1 = edge_index[1]
    batch3d = batch.reshape(N_RB, 1, RB)
    params = [(W1_0, b1_0, W2_0, b2_0), (W1_1, b1_1, W2_1, b2_1),
              (W1_2, b1_2, W2_2, b2_2)]
    h = x
    pooled = []
    for li, (W1, b1, W2, b2) in enumerate(params):
        agg = _sc_scatter(h, src2, dst1)
        prev = tuple(pooled) if li == 2 else None
        h, p = _tc_layer(h, agg, batch3d,
                         W1, b1.reshape(1, D), W2, b2.reshape(1, D),
                         prev_pooled=prev)
        pooled.append(p)
    return pooled[2]
